# baseline (speedup 1.0000x reference)
import functools

import jax
import jax.numpy as jnp
from jax import lax
from jax.experimental import pallas as pl
from jax.experimental.pallas import tpu as pltpu
from jax.experimental.pallas import tpu_sc as plsc

_BN_EPS = 1e-5
_NC = 2
_NS = 16
_BATCH = 128
_ROWS = 1024
_ZR = 16


def _mm_body(x_ref, w_ref, o_ref):
    o_ref[...] = jnp.dot(x_ref[...], w_ref[...],
                         preferred_element_type=jnp.float32)


def _linear_tc(x, W):
    M, K = x.shape
    F = W.shape[1]
    grid = M // _ROWS
    return pl.pallas_call(
        _mm_body,
        grid=(grid,),
        in_specs=[
            pl.BlockSpec((_ROWS, K), lambda i: (i, 0)),
            pl.BlockSpec((K, F), lambda i: (0, 0)),
        ],
        out_specs=pl.BlockSpec((_ROWS, F), lambda i: (i, 0)),
        out_shape=jax.ShapeDtypeStruct((M, F), jnp.float32),
    )(x, W)


def _segment_sum_sc(y, src_w, dst_w, n_pad, feat, nb):
    rows_per_tile = n_pad // _NS

    def body(y_hbm, src_hbm, dst_hbm, out_hbm, src_v, dst_v, rows0_v,
             rows1_v, zbuf_v, acc_sh, sem0, sem1):
        c = lax.axis_index("c")
        s = lax.axis_index("s")
        wid = s * _NC + c
        for r in range(_ZR):
            for q in range(feat // 16):
                zbuf_v[r, pl.ds(q * 16, 16)] = jnp.zeros((16,), jnp.float32)
        base = s * rows_per_tile

        def zloop(i, carry):
            pltpu.sync_copy(zbuf_v, acc_sh.at[pl.ds(base + i * _ZR, _ZR)])
            return carry

        lax.fori_loop(0, rows_per_tile // _ZR, zloop, 0)
        pltpu.sync_copy(src_hbm.at[wid], src_v)
        pltpu.sync_copy(dst_hbm.at[wid], dst_v)
        plsc.subcore_barrier()

        def fire(j, buf, sem):
            pltpu.async_copy(y_hbm.at[src_v.at[j]], buf, sem)

        def drain(j, buf, sem):
            pltpu.make_async_copy(y_hbm.at[src_v.at[j]], buf, sem).wait()

        def scat(j, buf):
            pltpu.sync_copy(buf, acc_sh.at[dst_v.at[j]], add=True)

        fire(0, rows0_v, sem0)

        def eloop(p, carry):
            j = 2 * p
            fire(j + 1, rows1_v, sem1)
            drain(j, rows0_v, sem0)
            scat(j, rows0_v)

            @pl.when(j + 2 < nb)
            def _():
                fire(j + 2, rows0_v, sem0)

            drain(j + 1, rows1_v, sem1)
            scat(j + 1, rows1_v)
            return carry

        lax.fori_loop(0, nb // 2, eloop, 0)
        plsc.subcore_barrier()
        pltpu.sync_copy(acc_sh.at[pl.ds(base, rows_per_tile)],
                        out_hbm.at[c, pl.ds(base, rows_per_tile)])

    k = pl.kernel(
        body,
        out_type=jax.ShapeDtypeStruct((_NC, n_pad, feat), jnp.float32),
        mesh=plsc.VectorSubcoreMesh(core_axis_name="c", subcore_axis_name="s"),
        scratch_types=[
            pltpu.VMEM((nb, _BATCH), jnp.int32),
            pltpu.VMEM((nb, _BATCH), jnp.int32),
            pltpu.VMEM((_BATCH, feat), jnp.float32),
            pltpu.VMEM((_BATCH, feat), jnp.float32),
            pltpu.VMEM((_ZR, feat), jnp.float32),
            pltpu.VMEM_SHARED((n_pad, feat), jnp.float32),
            pltpu.SemaphoreType.DMA,
            pltpu.SemaphoreType.DMA,
        ],
        compiler_params=pltpu.CompilerParams(use_tc_tiling_on_sc=False),
    )
    return k(y, src_w, dst_w)


def _block_body(y_ref, a0_ref, a1_ref, em_ref, b1_ref, w2_ref, b2_ref,
                g_ref, be_ref, w1n_ref, o_ref):
    z = em_ref[...] * y_ref[...] + a0_ref[...] + a1_ref[...] + b1_ref[...]
    z = jnp.maximum(z, 0.0)
    t = jnp.dot(z, w2_ref[...], preferred_element_type=jnp.float32) + b2_ref[...]
    h = jnp.maximum(g_ref[...] * t + be_ref[...], 0.0)
    o_ref[...] = jnp.dot(h, w1n_ref[...], preferred_element_type=jnp.float32)


def _mlp_block_tc(y, a0, a1, em, b1, W2, b2, g, be, W1n):
    M, H = y.shape
    F = W1n.shape[1]
    grid = M // _ROWS
    row = lambda i: (i, 0)
    one = lambda i: (0, 0)
    return pl.pallas_call(
        _block_body,
        grid=(grid,),
        in_specs=[
            pl.BlockSpec((_ROWS, H), row),
            pl.BlockSpec((_ROWS, H), row),
            pl.BlockSpec((_ROWS, H), row),
            pl.BlockSpec((1, H), one),
            pl.BlockSpec((1, H), one),
            pl.BlockSpec((H, H), one),
            pl.BlockSpec((1, H), one),
            pl.BlockSpec((1, H), one),
            pl.BlockSpec((1, H), one),
            pl.BlockSpec((H, F), one),
        ],
        out_specs=pl.BlockSpec((_ROWS, F), row),
        out_shape=jax.ShapeDtypeStruct((M, F), jnp.float32),
    )(y, a0, a1, em, b1, W2, b2, g, be, W1n)


def _final_body(n_real, y_ref, a0_ref, a1_ref, em_ref, b1_ref, w2_ref,
                b2_ref, g_ref, be_ref, o_ref, acc_ref):
    i = pl.program_id(0)
    z = em_ref[...] * y_ref[...] + a0_ref[...] + a1_ref[...] + b1_ref[...]
    z = jnp.maximum(z, 0.0)
    rowid = lax.broadcasted_iota(jnp.int32, z.shape, 0) + i * _ROWS
    z = jnp.where(rowid < n_real, z, 0.0)
    part = jnp.sum(z, axis=0, keepdims=True)

    @pl.when(i == 0)
    def _():
        acc_ref[...] = jnp.zeros_like(acc_ref)

    acc_ref[...] += part

    @pl.when(i == pl.num_programs(0) - 1)
    def _():
        s = acc_ref[...] * (1.0 / n_real)
        t = jnp.dot(s, w2_ref[...], preferred_element_type=jnp.float32)
        o_ref[...] = g_ref[...] * (t + b2_ref[...]) + be_ref[...]


def _final_tc(y, a0, a1, em, b1, W2, b2, g, be, n_real):
    M, F = y.shape
    grid = M // _ROWS
    row = lambda i: (i, 0)
    one = lambda i: (0, 0)
    return pl.pallas_call(
        functools.partial(_final_body, n_real),
        grid=(grid,),
        in_specs=[
            pl.BlockSpec((_ROWS, F), row),
            pl.BlockSpec((_ROWS, F), row),
            pl.BlockSpec((_ROWS, F), row),
            pl.BlockSpec((1, F), one),
            pl.BlockSpec((1, F), one),
            pl.BlockSpec((F, F), one),
            pl.BlockSpec((1, F), one),
            pl.BlockSpec((1, F), one),
            pl.BlockSpec((1, F), one),
        ],
        out_specs=pl.BlockSpec((1, F), one),
        out_shape=jax.ShapeDtypeStruct((1, F), jnp.float32),
        scratch_shapes=[pltpu.VMEM((1, F), jnp.float32)],
    )(y, a0, a1, em, b1, W2, b2, g, be)


def _round_up(v, m):
    return (v + m - 1) // m * m


def kernel(x, edge_index, W1_0, b1_0, W2_0, b2_0, eps0, gamma0, beta0,
           W1_1, b1_1, W2_1, b2_1, eps1, gamma1, beta1):
    N, _ = x.shape
    H = W1_0.shape[1]
    OUT = W1_1.shape[1]
    E = edge_index.shape[1]
    NW = _NC * _NS

    n_pad = _round_up(N + 1, _ROWS)
    nb = -(-E // (NW * _BATCH))
    nb += nb % 2
    e_pad = NW * _BATCH * nb

    x_p = jnp.pad(x.astype(jnp.float32), ((0, n_pad - N), (0, 0)))
    src = edge_index[0]
    dst = edge_index[1]
    src_w = jnp.concatenate(
        [src, jnp.zeros((e_pad - E,), jnp.int32)]).reshape(NW, nb, _BATCH)
    dst_w = jnp.concatenate(
        [dst, jnp.full((e_pad - E,), N, jnp.int32)]).reshape(NW, nb, _BATCH)

    bn_scale = 1.0 / jnp.sqrt(1.0 + _BN_EPS)
    em0 = (1.0 + eps0) * jnp.ones((1, H), jnp.float32)
    em1 = (1.0 + eps1) * jnp.ones((1, OUT), jnp.float32)
    g0 = (gamma0 * bn_scale).reshape(1, H)
    g1 = (gamma1 * bn_scale).reshape(1, OUT)

    y0 = _linear_tc(x_p, W1_0)
    acc0 = _segment_sum_sc(y0, src_w, dst_w, n_pad, H, nb)
    y1 = _mlp_block_tc(y0, acc0[0], acc0[1], em0, b1_0.reshape(1, H),
                       W2_0, b2_0.reshape(1, H), g0, beta0.reshape(1, H),
                       W1_1)
    acc1 = _segment_sum_sc(y1, src_w, dst_w, n_pad, OUT, nb)
    return _final_tc(y1, acc1[0], acc1[1], em1, b1_1.reshape(1, OUT),
                     W2_1, b2_1.reshape(1, OUT), g1, beta1.reshape(1, OUT),
                     N)


# speedup vs baseline: 1.7721x; 1.7721x over previous
import functools

import jax
import jax.numpy as jnp
from jax import lax
from jax.experimental import pallas as pl
from jax.experimental.pallas import tpu as pltpu
from jax.experimental.pallas import tpu_sc as plsc

_BN_EPS = 1e-5
_NC = 2
_NS = 16
_BATCH = 128
_ROWS = 1024
_ZR = 16


def _mm_body(x_ref, w_ref, o_ref):
    o_ref[...] = jnp.dot(x_ref[...], w_ref[...],
                         preferred_element_type=jnp.float32)


def _linear_tc(x, W):
    M, K = x.shape
    F = W.shape[1]
    grid = M // _ROWS
    return pl.pallas_call(
        _mm_body,
        grid=(grid,),
        in_specs=[
            pl.BlockSpec((_ROWS, K), lambda i: (i, 0)),
            pl.BlockSpec((K, F), lambda i: (0, 0)),
        ],
        out_specs=pl.BlockSpec((_ROWS, F), lambda i: (i, 0)),
        out_shape=jax.ShapeDtypeStruct((M, F), jnp.float32),
    )(x, W)


def _segment_sum_sc(y, src_w, dst_w, n_pad, feat, nb):
    rows_per_tile = n_pad // _NS

    def body(y_hbm, src_hbm, dst_hbm, out_hbm, src_v, dst_v, rows0_v,
             rows1_v, zbuf_v, y_sh, acc_sh, sem0, sem1):
        c = lax.axis_index("c")
        s = lax.axis_index("s")
        wid = s * _NC + c
        for r in range(_ZR):
            for q in range(feat // 16):
                zbuf_v[r, pl.ds(q * 16, 16)] = jnp.zeros((16,), jnp.float32)
        base = s * rows_per_tile

        def zloop(i, carry):
            pltpu.sync_copy(zbuf_v, acc_sh.at[pl.ds(base + i * _ZR, _ZR)])
            return carry

        lax.fori_loop(0, rows_per_tile // _ZR, zloop, 0)
        pltpu.sync_copy(y_hbm.at[pl.ds(base, rows_per_tile)],
                        y_sh.at[pl.ds(base, rows_per_tile)])
        pltpu.sync_copy(src_hbm.at[wid], src_v)
        pltpu.sync_copy(dst_hbm.at[wid], dst_v)
        plsc.subcore_barrier()

        def fire(j, buf, sem):
            pltpu.async_copy(y_sh.at[src_v.at[j]], buf, sem)

        def drain(j, buf, sem):
            pltpu.make_async_copy(y_sh.at[src_v.at[j]], buf, sem).wait()

        def scat(j, buf):
            pltpu.sync_copy(buf, acc_sh.at[dst_v.at[j]], add=True)

        fire(0, rows0_v, sem0)

        def eloop(p, carry):
            j = 2 * p
            fire(j + 1, rows1_v, sem1)
            drain(j, rows0_v, sem0)
            scat(j, rows0_v)

            @pl.when(j + 2 < nb)
            def _():
                fire(j + 2, rows0_v, sem0)

            drain(j + 1, rows1_v, sem1)
            scat(j + 1, rows1_v)
            return carry

        lax.fori_loop(0, nb // 2, eloop, 0)
        plsc.subcore_barrier()
        pltpu.sync_copy(acc_sh.at[pl.ds(base, rows_per_tile)],
                        out_hbm.at[c, pl.ds(base, rows_per_tile)])

    k = pl.kernel(
        body,
        out_type=jax.ShapeDtypeStruct((_NC, n_pad, feat), jnp.float32),
        mesh=plsc.VectorSubcoreMesh(core_axis_name="c", subcore_axis_name="s"),
        scratch_types=[
            pltpu.VMEM((nb, _BATCH), jnp.int32),
            pltpu.VMEM((nb, _BATCH), jnp.int32),
            pltpu.VMEM((_BATCH, feat), jnp.float32),
            pltpu.VMEM((_BATCH, feat), jnp.float32),
            pltpu.VMEM((_ZR, feat), jnp.float32),
            pltpu.VMEM_SHARED((n_pad, feat), jnp.float32),
            pltpu.VMEM_SHARED((n_pad, feat), jnp.float32),
            pltpu.SemaphoreType.DMA,
            pltpu.SemaphoreType.DMA,
        ],
        compiler_params=pltpu.CompilerParams(use_tc_tiling_on_sc=False),
    )
    return k(y, src_w, dst_w)


def _block_body(y_ref, a0_ref, a1_ref, em_ref, b1_ref, w2_ref, b2_ref,
                g_ref, be_ref, w1n_ref, o_ref):
    z = em_ref[...] * y_ref[...] + a0_ref[...] + a1_ref[...] + b1_ref[...]
    z = jnp.maximum(z, 0.0)
    t = jnp.dot(z, w2_ref[...], preferred_element_type=jnp.float32) + b2_ref[...]
    h = jnp.maximum(g_ref[...] * t + be_ref[...], 0.0)
    o_ref[...] = jnp.dot(h, w1n_ref[...], preferred_element_type=jnp.float32)


def _mlp_block_tc(y, a0, a1, em, b1, W2, b2, g, be, W1n):
    M, H = y.shape
    F = W1n.shape[1]
    grid = M // _ROWS
    row = lambda i: (i, 0)
    one = lambda i: (0, 0)
    return pl.pallas_call(
        _block_body,
        grid=(grid,),
        in_specs=[
            pl.BlockSpec((_ROWS, H), row),
            pl.BlockSpec((_ROWS, H), row),
            pl.BlockSpec((_ROWS, H), row),
            pl.BlockSpec((1, H), one),
            pl.BlockSpec((1, H), one),
            pl.BlockSpec((H, H), one),
            pl.BlockSpec((1, H), one),
            pl.BlockSpec((1, H), one),
            pl.BlockSpec((1, H), one),
            pl.BlockSpec((H, F), one),
        ],
        out_specs=pl.BlockSpec((_ROWS, F), row),
        out_shape=jax.ShapeDtypeStruct((M, F), jnp.float32),
    )(y, a0, a1, em, b1, W2, b2, g, be, W1n)


def _final_body(n_real, y_ref, a0_ref, a1_ref, em_ref, b1_ref, w2_ref,
                b2_ref, g_ref, be_ref, o_ref, acc_ref):
    i = pl.program_id(0)
    z = em_ref[...] * y_ref[...] + a0_ref[...] + a1_ref[...] + b1_ref[...]
    z = jnp.maximum(z, 0.0)
    rowid = lax.broadcasted_iota(jnp.int32, z.shape, 0) + i * _ROWS
    z = jnp.where(rowid < n_real, z, 0.0)
    part = jnp.sum(z, axis=0, keepdims=True)

    @pl.when(i == 0)
    def _():
        acc_ref[...] = jnp.zeros_like(acc_ref)

    acc_ref[...] += part

    @pl.when(i == pl.num_programs(0) - 1)
    def _():
        s = acc_ref[...] * (1.0 / n_real)
        t = jnp.dot(s, w2_ref[...], preferred_element_type=jnp.float32)
        o_ref[...] = g_ref[...] * (t + b2_ref[...]) + be_ref[...]


def _final_tc(y, a0, a1, em, b1, W2, b2, g, be, n_real):
    M, F = y.shape
    grid = M // _ROWS
    row = lambda i: (i, 0)
    one = lambda i: (0, 0)
    return pl.pallas_call(
        functools.partial(_final_body, n_real),
        grid=(grid,),
        in_specs=[
            pl.BlockSpec((_ROWS, F), row),
            pl.BlockSpec((_ROWS, F), row),
            pl.BlockSpec((_ROWS, F), row),
            pl.BlockSpec((1, F), one),
            pl.BlockSpec((1, F), one),
            pl.BlockSpec((F, F), one),
            pl.BlockSpec((1, F), one),
            pl.BlockSpec((1, F), one),
            pl.BlockSpec((1, F), one),
        ],
        out_specs=pl.BlockSpec((1, F), one),
        out_shape=jax.ShapeDtypeStruct((1, F), jnp.float32),
        scratch_shapes=[pltpu.VMEM((1, F), jnp.float32)],
    )(y, a0, a1, em, b1, W2, b2, g, be)


def _round_up(v, m):
    return (v + m - 1) // m * m


def kernel(x, edge_index, W1_0, b1_0, W2_0, b2_0, eps0, gamma0, beta0,
           W1_1, b1_1, W2_1, b2_1, eps1, gamma1, beta1):
    N, _ = x.shape
    H = W1_0.shape[1]
    OUT = W1_1.shape[1]
    E = edge_index.shape[1]
    NW = _NC * _NS

    n_pad = _round_up(N + 1, _ROWS)
    nb = -(-E // (NW * _BATCH))
    nb += nb % 2
    e_pad = NW * _BATCH * nb

    x_p = jnp.pad(x.astype(jnp.float32), ((0, n_pad - N), (0, 0)))
    src = edge_index[0]
    dst = edge_index[1]
    src_w = jnp.concatenate(
        [src, jnp.zeros((e_pad - E,), jnp.int32)]).reshape(NW, nb, _BATCH)
    dst_w = jnp.concatenate(
        [dst, jnp.full((e_pad - E,), N, jnp.int32)]).reshape(NW, nb, _BATCH)

    bn_scale = 1.0 / jnp.sqrt(1.0 + _BN_EPS)
    em0 = (1.0 + eps0) * jnp.ones((1, H), jnp.float32)
    em1 = (1.0 + eps1) * jnp.ones((1, OUT), jnp.float32)
    g0 = (gamma0 * bn_scale).reshape(1, H)
    g1 = (gamma1 * bn_scale).reshape(1, OUT)

    y0 = _linear_tc(x_p, W1_0)
    acc0 = _segment_sum_sc(y0, src_w, dst_w, n_pad, H, nb)
    y1 = _mlp_block_tc(y0, acc0[0], acc0[1], em0, b1_0.reshape(1, H),
                       W2_0, b2_0.reshape(1, H), g0, beta0.reshape(1, H),
                       W1_1)
    acc1 = _segment_sum_sc(y1, src_w, dst_w, n_pad, OUT, nb)
    return _final_tc(y1, acc1[0], acc1[1], em1, b1_1.reshape(1, OUT),
                     W2_1, b2_1.reshape(1, OUT), g1, beta1.reshape(1, OUT),
                     N)


# speedup vs baseline: 1.9364x; 1.0928x over previous
import functools

import jax
import jax.numpy as jnp
from jax import lax
from jax.experimental import pallas as pl
from jax.experimental.pallas import tpu as pltpu
from jax.experimental.pallas import tpu_sc as plsc

_BN_EPS = 1e-5
_NC = 2
_NS = 16
_BATCH = 128
_ROWS = 1024
_ZR = 16


def _mm_body(x_ref, w_ref, o_ref):
    o_ref[...] = jnp.dot(x_ref[...], w_ref[...],
                         preferred_element_type=jnp.float32)


def _linear_tc(x, W, m_out):
    _, K = x.shape
    F = W.shape[1]
    grid = m_out // _ROWS
    return pl.pallas_call(
        _mm_body,
        grid=(grid,),
        in_specs=[
            pl.BlockSpec((_ROWS, K), lambda i: (i, 0)),
            pl.BlockSpec((K, F), lambda i: (0, 0)),
        ],
        out_specs=pl.BlockSpec((_ROWS, F), lambda i: (i, 0)),
        out_shape=jax.ShapeDtypeStruct((m_out, F), jnp.float32),
    )(x, W)


def _segment_sum_sc(y, src_w, dst_w, n_pad, feat, nb):
    rows_per_tile = n_pad // _NS

    def body(y_hbm, src_hbm, dst_hbm, out_hbm, src_v, dst_v, rows0_v,
             rows1_v, zbuf_v, y_sh, acc_sh, sem0, sem1):
        c = lax.axis_index("c")
        s = lax.axis_index("s")
        wid = s * _NC + c
        for r in range(_ZR):
            for q in range(feat // 16):
                zbuf_v[r, pl.ds(q * 16, 16)] = jnp.zeros((16,), jnp.float32)
        base = s * rows_per_tile

        def zloop(i, carry):
            pltpu.sync_copy(zbuf_v, acc_sh.at[pl.ds(base + i * _ZR, _ZR)])
            return carry

        lax.fori_loop(0, rows_per_tile // _ZR, zloop, 0)
        pltpu.sync_copy(y_hbm.at[pl.ds(base, rows_per_tile)],
                        y_sh.at[pl.ds(base, rows_per_tile)])
        pltpu.sync_copy(src_hbm.at[wid], src_v)
        pltpu.sync_copy(dst_hbm.at[wid], dst_v)
        plsc.subcore_barrier()

        def fire(j, buf, sem):
            pltpu.async_copy(y_sh.at[src_v.at[j]], buf, sem)

        def drain(j, buf, sem):
            pltpu.make_async_copy(y_sh.at[src_v.at[j]], buf, sem).wait()

        def scat(j, buf):
            pltpu.sync_copy(buf, acc_sh.at[dst_v.at[j]], add=True)

        fire(0, rows0_v, sem0)

        def eloop(p, carry):
            j = 2 * p
            fire(j + 1, rows1_v, sem1)
            drain(j, rows0_v, sem0)
            scat(j, rows0_v)

            @pl.when(j + 2 < nb)
            def _():
                fire(j + 2, rows0_v, sem0)

            drain(j + 1, rows1_v, sem1)
            scat(j + 1, rows1_v)
            return carry

        lax.fori_loop(0, nb // 2, eloop, 0)
        plsc.subcore_barrier()
        pltpu.sync_copy(acc_sh.at[pl.ds(base, rows_per_tile)],
                        out_hbm.at[c, pl.ds(base, rows_per_tile)])

    k = pl.kernel(
        body,
        out_type=jax.ShapeDtypeStruct((_NC, n_pad, feat), jnp.float32),
        mesh=plsc.VectorSubcoreMesh(core_axis_name="c", subcore_axis_name="s"),
        scratch_types=[
            pltpu.VMEM((nb, _BATCH), jnp.int32),
            pltpu.VMEM((nb, _BATCH), jnp.int32),
            pltpu.VMEM((_BATCH, feat), jnp.float32),
            pltpu.VMEM((_BATCH, feat), jnp.float32),
            pltpu.VMEM((_ZR, feat), jnp.float32),
            pltpu.VMEM_SHARED((n_pad, feat), jnp.float32),
            pltpu.VMEM_SHARED((n_pad, feat), jnp.float32),
            pltpu.SemaphoreType.DMA,
            pltpu.SemaphoreType.DMA,
        ],
        compiler_params=pltpu.CompilerParams(use_tc_tiling_on_sc=False),
    )
    return k(y, src_w, dst_w)


def _block_body(y_ref, a_ref, em_ref, b1_ref, w2_ref, b2_ref,
                g_ref, be_ref, w1n_ref, o_ref):
    z = em_ref[...] * y_ref[...] + a_ref[0] + a_ref[1] + b1_ref[...]
    z = jnp.maximum(z, 0.0)
    t = jnp.dot(z, w2_ref[...], preferred_element_type=jnp.float32) + b2_ref[...]
    h = jnp.maximum(g_ref[...] * t + be_ref[...], 0.0)
    o_ref[...] = jnp.dot(h, w1n_ref[...], preferred_element_type=jnp.float32)


def _mlp_block_tc(y, acc, em, b1, W2, b2, g, be, W1n):
    M, H = y.shape
    F = W1n.shape[1]
    grid = M // _ROWS
    row = lambda i: (i, 0)
    one = lambda i: (0, 0)
    return pl.pallas_call(
        _block_body,
        grid=(grid,),
        in_specs=[
            pl.BlockSpec((_ROWS, H), row),
            pl.BlockSpec((2, _ROWS, H), lambda i: (0, i, 0)),
            pl.BlockSpec((1, H), one),
            pl.BlockSpec((1, H), one),
            pl.BlockSpec((H, H), one),
            pl.BlockSpec((1, H), one),
            pl.BlockSpec((1, H), one),
            pl.BlockSpec((1, H), one),
            pl.BlockSpec((H, F), one),
        ],
        out_specs=pl.BlockSpec((_ROWS, F), row),
        out_shape=jax.ShapeDtypeStruct((M, F), jnp.float32),
    )(y, acc, em, b1, W2, b2, g, be, W1n)


def _final_body(n_real, y_ref, a_ref, em_ref, b1_ref, w2_ref,
                b2_ref, g_ref, be_ref, o_ref, acc_ref):
    i = pl.program_id(0)
    z = em_ref[...] * y_ref[...] + a_ref[0] + a_ref[1] + b1_ref[...]
    z = jnp.maximum(z, 0.0)
    rowid = lax.broadcasted_iota(jnp.int32, z.shape, 0) + i * _ROWS
    z = jnp.where(rowid < n_real, z, 0.0)
    part = jnp.sum(z, axis=0, keepdims=True)

    @pl.when(i == 0)
    def _():
        acc_ref[...] = jnp.zeros_like(acc_ref)

    acc_ref[...] += part

    @pl.when(i == pl.num_programs(0) - 1)
    def _():
        s = acc_ref[...] * (1.0 / n_real)
        t = jnp.dot(s, w2_ref[...], preferred_element_type=jnp.float32)
        o_ref[...] = g_ref[...] * (t + b2_ref[...]) + be_ref[...]


def _final_tc(y, acc, em, b1, W2, b2, g, be, n_real):
    M, F = y.shape
    grid = M // _ROWS
    row = lambda i: (i, 0)
    one = lambda i: (0, 0)
    return pl.pallas_call(
        functools.partial(_final_body, n_real),
        grid=(grid,),
        in_specs=[
            pl.BlockSpec((_ROWS, F), row),
            pl.BlockSpec((2, _ROWS, F), lambda i: (0, i, 0)),
            pl.BlockSpec((1, F), one),
            pl.BlockSpec((1, F), one),
            pl.BlockSpec((F, F), one),
            pl.BlockSpec((1, F), one),
            pl.BlockSpec((1, F), one),
            pl.BlockSpec((1, F), one),
        ],
        out_specs=pl.BlockSpec((1, F), one),
        out_shape=jax.ShapeDtypeStruct((1, F), jnp.float32),
        scratch_shapes=[pltpu.VMEM((1, F), jnp.float32)],
    )(y, acc, em, b1, W2, b2, g, be)


def _round_up(v, m):
    return (v + m - 1) // m * m


def kernel(x, edge_index, W1_0, b1_0, W2_0, b2_0, eps0, gamma0, beta0,
           W1_1, b1_1, W2_1, b2_1, eps1, gamma1, beta1):
    N, _ = x.shape
    H = W1_0.shape[1]
    OUT = W1_1.shape[1]
    E = edge_index.shape[1]
    NW = _NC * _NS

    n_pad = _round_up(N + 1, _ROWS)
    nb = -(-E // (NW * _BATCH))
    nb += nb % 2
    e_pad = NW * _BATCH * nb

    src = edge_index[0]
    dst = edge_index[1]
    src_w = jnp.concatenate(
        [src, jnp.zeros((e_pad - E,), jnp.int32)]).reshape(NW, nb, _BATCH)
    dst_w = jnp.concatenate(
        [dst, jnp.full((e_pad - E,), N, jnp.int32)]).reshape(NW, nb, _BATCH)

    bn_scale = 1.0 / jnp.sqrt(1.0 + _BN_EPS)
    em0 = (1.0 + eps0) * jnp.ones((1, H), jnp.float32)
    em1 = (1.0 + eps1) * jnp.ones((1, OUT), jnp.float32)
    g0 = (gamma0 * bn_scale).reshape(1, H)
    g1 = (gamma1 * bn_scale).reshape(1, OUT)

    y0 = _linear_tc(x, W1_0, n_pad)
    acc0 = _segment_sum_sc(y0, src_w, dst_w, n_pad, H, nb)
    y1 = _mlp_block_tc(y0, acc0, em0, b1_0.reshape(1, H),
                       W2_0, b2_0.reshape(1, H), g0, beta0.reshape(1, H),
                       W1_1)
    acc1 = _segment_sum_sc(y1, src_w, dst_w, n_pad, OUT, nb)
    return _final_tc(y1, acc1, em1, b1_1.reshape(1, OUT),
                     W2_1, b2_1.reshape(1, OUT), g1, beta1.reshape(1, OUT),
                     N)


# speedup vs baseline: 2.0784x; 1.0733x over previous
import functools

import jax
import jax.numpy as jnp
from jax import lax
from jax.experimental import pallas as pl
from jax.experimental.pallas import tpu as pltpu
from jax.experimental.pallas import tpu_sc as plsc

_BN_EPS = 1e-5
_NC = 2
_NS = 16
_BATCH = 128
_ROWS = 1024
_ZR = 16


def _mm_body(x_ref, w_ref, o_ref):
    o_ref[...] = jnp.dot(x_ref[...], w_ref[...],
                         preferred_element_type=jnp.float32)


def _linear_tc(x, W, m_out):
    _, K = x.shape
    F = W.shape[1]
    grid = m_out // _ROWS
    return pl.pallas_call(
        _mm_body,
        grid=(grid,),
        in_specs=[
            pl.BlockSpec((_ROWS, K), lambda i: (i, 0)),
            pl.BlockSpec((K, F), lambda i: (0, 0)),
        ],
        out_specs=pl.BlockSpec((_ROWS, F), lambda i: (i, 0)),
        out_shape=jax.ShapeDtypeStruct((m_out, F), jnp.float32),
    )(x, W)


def _segment_sum_sc(y, eidx, n_pad, feat):
    rows_per_tile = n_pad // _NS
    nbat = eidx.shape[1]
    nw = _NC * _NS
    q, rem = divmod(nbat, nw)

    def body(y_hbm, eidx_hbm, out_hbm, src_v, dst_v, rows0_v,
             rows1_v, zbuf_v, y_sh, acc_sh, sem0, sem1):
        c = lax.axis_index("c")
        s = lax.axis_index("s")
        wid = s * _NC + c
        for r in range(_ZR):
            for qq in range(feat // 16):
                zbuf_v[r, pl.ds(qq * 16, 16)] = jnp.zeros((16,), jnp.float32)
        base = s * rows_per_tile

        def zloop(i, carry):
            pltpu.sync_copy(zbuf_v, acc_sh.at[pl.ds(base + i * _ZR, _ZR)])
            return carry

        lax.fori_loop(0, rows_per_tile // _ZR, zloop, 0)
        pltpu.sync_copy(y_hbm.at[pl.ds(base, rows_per_tile)],
                        y_sh.at[pl.ds(base, rows_per_tile)])
        extra = (wid < rem).astype(jnp.int32)
        nb_w = q + extra
        base_b = wid * q + jnp.minimum(wid, rem)
        pltpu.sync_copy(eidx_hbm.at[0, pl.ds(base_b, q)],
                        src_v.at[pl.ds(0, q)])
        pltpu.sync_copy(eidx_hbm.at[1, pl.ds(base_b, q)],
                        dst_v.at[pl.ds(0, q)])

        @pl.when(extra == 1)
        def _():
            pltpu.sync_copy(eidx_hbm.at[0, pl.ds(base_b + q, 1)],
                            src_v.at[pl.ds(q, 1)])
            pltpu.sync_copy(eidx_hbm.at[1, pl.ds(base_b + q, 1)],
                            dst_v.at[pl.ds(q, 1)])

        plsc.subcore_barrier()

        def fire(j, buf, sem):
            pltpu.async_copy(y_sh.at[src_v.at[j]], buf, sem)

        def drain(j, buf, sem):
            pltpu.make_async_copy(y_sh.at[src_v.at[j]], buf, sem).wait()

        def scat(j, buf):
            pltpu.sync_copy(buf, acc_sh.at[dst_v.at[j]], add=True)

        def step(j, buf, sem, obuf, osem):
            @pl.when(j + 1 < nb_w)
            def _():
                fire(j + 1, obuf, osem)

            drain(j, buf, sem)
            scat(j, buf)

        fire(0, rows0_v, sem0)

        def eloop2(p, carry):
            j = 2 * p
            step(j, rows0_v, sem0, rows1_v, sem1)
            step(j + 1, rows1_v, sem1, rows0_v, sem0)
            return carry

        lax.fori_loop(0, nb_w // 2, eloop2, 0)

        @pl.when(nb_w % 2 == 1)
        def _():
            step(nb_w - 1, rows0_v, sem0, rows1_v, sem1)

        plsc.subcore_barrier()
        pltpu.sync_copy(acc_sh.at[pl.ds(base, rows_per_tile)],
                        out_hbm.at[c, pl.ds(base, rows_per_tile)])

    k = pl.kernel(
        body,
        out_type=jax.ShapeDtypeStruct((_NC, n_pad, feat), jnp.float32),
        mesh=plsc.VectorSubcoreMesh(core_axis_name="c", subcore_axis_name="s"),
        scratch_types=[
            pltpu.VMEM((q + 1, _BATCH), jnp.int32),
            pltpu.VMEM((q + 1, _BATCH), jnp.int32),
            pltpu.VMEM((_BATCH, feat), jnp.float32),
            pltpu.VMEM((_BATCH, feat), jnp.float32),
            pltpu.VMEM((_ZR, feat), jnp.float32),
            pltpu.VMEM_SHARED((n_pad, feat), jnp.float32),
            pltpu.VMEM_SHARED((n_pad, feat), jnp.float32),
            pltpu.SemaphoreType.DMA,
            pltpu.SemaphoreType.DMA,
        ],
        compiler_params=pltpu.CompilerParams(use_tc_tiling_on_sc=False),
    )
    return k(y, eidx)


def _block_body(y_ref, a_ref, em_ref, b1_ref, w2_ref, b2_ref,
                g_ref, be_ref, w1n_ref, o_ref):
    z = em_ref[...] * y_ref[...] + a_ref[0] + a_ref[1] + b1_ref[...]
    z = jnp.maximum(z, 0.0)
    t = jnp.dot(z, w2_ref[...], preferred_element_type=jnp.float32) + b2_ref[...]
    h = jnp.maximum(g_ref[...] * t + be_ref[...], 0.0)
    o_ref[...] = jnp.dot(h, w1n_ref[...], preferred_element_type=jnp.float32)


def _mlp_block_tc(y, acc, em, b1, W2, b2, g, be, W1n):
    M, H = y.shape
    F = W1n.shape[1]
    grid = M // _ROWS
    row = lambda i: (i, 0)
    one = lambda i: (0, 0)
    return pl.pallas_call(
        _block_body,
        grid=(grid,),
        in_specs=[
            pl.BlockSpec((_ROWS, H), row),
            pl.BlockSpec((2, _ROWS, H), lambda i: (0, i, 0)),
            pl.BlockSpec((1, H), one),
            pl.BlockSpec((1, H), one),
            pl.BlockSpec((H, H), one),
            pl.BlockSpec((1, H), one),
            pl.BlockSpec((1, H), one),
            pl.BlockSpec((1, H), one),
            pl.BlockSpec((H, F), one),
        ],
        out_specs=pl.BlockSpec((_ROWS, F), row),
        out_shape=jax.ShapeDtypeStruct((M, F), jnp.float32),
    )(y, acc, em, b1, W2, b2, g, be, W1n)


def _final_body(n_real, y_ref, a_ref, em_ref, b1_ref, w2_ref,
                b2_ref, g_ref, be_ref, o_ref, acc_ref):
    i = pl.program_id(0)
    z = em_ref[...] * y_ref[...] + a_ref[0] + a_ref[1] + b1_ref[...]
    z = jnp.maximum(z, 0.0)
    rowid = lax.broadcasted_iota(jnp.int32, z.shape, 0) + i * _ROWS
    z = jnp.where(rowid < n_real, z, 0.0)
    part = jnp.sum(z, axis=0, keepdims=True)

    @pl.when(i == 0)
    def _():
        acc_ref[...] = jnp.zeros_like(acc_ref)

    acc_ref[...] += part

    @pl.when(i == pl.num_programs(0) - 1)
    def _():
        s = acc_ref[...] * (1.0 / n_real)
        t = jnp.dot(s, w2_ref[...], preferred_element_type=jnp.float32)
        o_ref[...] = g_ref[...] * (t + b2_ref[...]) + be_ref[...]


def _final_tc(y, acc, em, b1, W2, b2, g, be, n_real):
    M, F = y.shape
    grid = M // _ROWS
    row = lambda i: (i, 0)
    one = lambda i: (0, 0)
    return pl.pallas_call(
        functools.partial(_final_body, n_real),
        grid=(grid,),
        in_specs=[
            pl.BlockSpec((_ROWS, F), row),
            pl.BlockSpec((2, _ROWS, F), lambda i: (0, i, 0)),
            pl.BlockSpec((1, F), one),
            pl.BlockSpec((1, F), one),
            pl.BlockSpec((F, F), one),
            pl.BlockSpec((1, F), one),
            pl.BlockSpec((1, F), one),
            pl.BlockSpec((1, F), one),
        ],
        out_specs=pl.BlockSpec((1, F), one),
        out_shape=jax.ShapeDtypeStruct((1, F), jnp.float32),
        scratch_shapes=[pltpu.VMEM((1, F), jnp.float32)],
    )(y, acc, em, b1, W2, b2, g, be)


def _round_up(v, m):
    return (v + m - 1) // m * m


def kernel(x, edge_index, W1_0, b1_0, W2_0, b2_0, eps0, gamma0, beta0,
           W1_1, b1_1, W2_1, b2_1, eps1, gamma1, beta1):
    N, _ = x.shape
    H = W1_0.shape[1]
    OUT = W1_1.shape[1]
    E = edge_index.shape[1]
    NW = _NC * _NS

    del NW
    n_pad = _round_up(N, _ROWS)
    assert E % _BATCH == 0, "edge count must divide the 128-edge batch"
    eidx = edge_index.reshape(2, E // _BATCH, _BATCH)

    bn_scale = 1.0 / jnp.sqrt(1.0 + _BN_EPS)
    em0 = (1.0 + eps0) * jnp.ones((1, H), jnp.float32)
    em1 = (1.0 + eps1) * jnp.ones((1, OUT), jnp.float32)
    g0 = (gamma0 * bn_scale).reshape(1, H)
    g1 = (gamma1 * bn_scale).reshape(1, OUT)

    y0 = _linear_tc(x, W1_0, n_pad)
    acc0 = _segment_sum_sc(y0, eidx, n_pad, H)
    y1 = _mlp_block_tc(y0, acc0, em0, b1_0.reshape(1, H),
                       W2_0, b2_0.reshape(1, H), g0, beta0.reshape(1, H),
                       W1_1)
    acc1 = _segment_sum_sc(y1, eidx, n_pad, OUT)
    return _final_tc(y1, acc1, em1, b1_1.reshape(1, OUT),
                     W2_1, b2_1.reshape(1, OUT), g1, beta1.reshape(1, OUT),
                     N)


# speedup vs baseline: 2.7865x; 1.3407x over previous
import functools

import jax
import jax.numpy as jnp
from jax import lax
from jax.experimental import pallas as pl
from jax.experimental.pallas import tpu as pltpu
from jax.experimental.pallas import tpu_sc as plsc

_BN_EPS = 1e-5
_NC = 2
_NS = 16
_BATCH = 128
_ROWS = 1024
_ZR = 16


def _mm_body(x_ref, w_ref, o_ref):
    o_ref[...] = jnp.dot(x_ref[...], w_ref[...],
                         preferred_element_type=jnp.float32
                         ).astype(jnp.bfloat16)


def _linear_tc(x, W, m_out):
    _, K = x.shape
    F = W.shape[1]
    grid = m_out // _ROWS
    return pl.pallas_call(
        _mm_body,
        grid=(grid,),
        in_specs=[
            pl.BlockSpec((_ROWS, K), lambda i: (i, 0)),
            pl.BlockSpec((K, F), lambda i: (0, 0)),
        ],
        out_specs=pl.BlockSpec((_ROWS, F), lambda i: (i, 0)),
        out_shape=jax.ShapeDtypeStruct((m_out, F), jnp.bfloat16),
    )(x, W)


def _segment_sum_sc(y, eidx, n_pad, feat):
    rows_per_tile = n_pad // _NS
    nbat = eidx.shape[1]
    nw = _NC * _NS
    q, rem = divmod(nbat, nw)

    def body(y_hbm, eidx_hbm, out_hbm, src_v, dst_v, rows0_v,
             rows1_v, zbuf_v, y_sh, acc_sh, sem0, sem1):
        c = lax.axis_index("c")
        s = lax.axis_index("s")
        wid = s * _NC + c
        for r in range(_ZR):
            for qq in range(feat // 32):
                zbuf_v[r, pl.ds(qq * 32, 32)] = jnp.zeros((32,), jnp.bfloat16)
        base = s * rows_per_tile

        def zloop(i, carry):
            pltpu.sync_copy(zbuf_v, acc_sh.at[pl.ds(base + i * _ZR, _ZR)])
            return carry

        lax.fori_loop(0, rows_per_tile // _ZR, zloop, 0)
        pltpu.sync_copy(y_hbm.at[pl.ds(base, rows_per_tile)],
                        y_sh.at[pl.ds(base, rows_per_tile)])
        extra = (wid < rem).astype(jnp.int32)
        nb_w = q + extra
        base_b = wid * q + jnp.minimum(wid, rem)
        pltpu.sync_copy(eidx_hbm.at[0, pl.ds(base_b, q)],
                        src_v.at[pl.ds(0, q)])
        pltpu.sync_copy(eidx_hbm.at[1, pl.ds(base_b, q)],
                        dst_v.at[pl.ds(0, q)])

        @pl.when(extra == 1)
        def _():
            pltpu.sync_copy(eidx_hbm.at[0, pl.ds(base_b + q, 1)],
                            src_v.at[pl.ds(q, 1)])
            pltpu.sync_copy(eidx_hbm.at[1, pl.ds(base_b + q, 1)],
                            dst_v.at[pl.ds(q, 1)])

        plsc.subcore_barrier()

        def fire(j, buf, sem):
            pltpu.async_copy(y_sh.at[src_v.at[j]], buf, sem)

        def drain(j, buf, sem):
            pltpu.make_async_copy(y_sh.at[src_v.at[j]], buf, sem).wait()

        def scat(j, buf):
            pltpu.sync_copy(buf, acc_sh.at[dst_v.at[j]], add=True)

        def step(j, buf, sem, obuf, osem):
            @pl.when(j + 1 < nb_w)
            def _():
                fire(j + 1, obuf, osem)

            drain(j, buf, sem)
            scat(j, buf)

        fire(0, rows0_v, sem0)

        def eloop2(p, carry):
            j = 2 * p
            step(j, rows0_v, sem0, rows1_v, sem1)
            step(j + 1, rows1_v, sem1, rows0_v, sem0)
            return carry

        lax.fori_loop(0, nb_w // 2, eloop2, 0)

        @pl.when(nb_w % 2 == 1)
        def _():
            step(nb_w - 1, rows0_v, sem0, rows1_v, sem1)

        plsc.subcore_barrier()
        pltpu.sync_copy(acc_sh.at[pl.ds(base, rows_per_tile)],
                        out_hbm.at[c, pl.ds(base, rows_per_tile)])

    k = pl.kernel(
        body,
        out_type=jax.ShapeDtypeStruct((_NC, n_pad, feat), jnp.bfloat16),
        mesh=plsc.VectorSubcoreMesh(core_axis_name="c", subcore_axis_name="s"),
        scratch_types=[
            pltpu.VMEM((q + 1, _BATCH), jnp.int32),
            pltpu.VMEM((q + 1, _BATCH), jnp.int32),
            pltpu.VMEM((_BATCH, feat), jnp.bfloat16),
            pltpu.VMEM((_BATCH, feat), jnp.bfloat16),
            pltpu.VMEM((_ZR, feat), jnp.bfloat16),
            pltpu.VMEM_SHARED((n_pad, feat), jnp.bfloat16),
            pltpu.VMEM_SHARED((n_pad, feat), jnp.bfloat16),
            pltpu.SemaphoreType.DMA,
            pltpu.SemaphoreType.DMA,
        ],
        compiler_params=pltpu.CompilerParams(use_tc_tiling_on_sc=False),
    )
    return k(y, eidx)


def _block_body(y_ref, a_ref, em_ref, b1_ref, w2_ref, b2_ref,
                g_ref, be_ref, w1n_ref, o_ref):
    agg = a_ref[0].astype(jnp.float32) + a_ref[1].astype(jnp.float32)
    z = em_ref[...] * y_ref[...].astype(jnp.float32) + agg + b1_ref[...]
    z = jnp.maximum(z, 0.0)
    t = jnp.dot(z, w2_ref[...], preferred_element_type=jnp.float32) + b2_ref[...]
    h = jnp.maximum(g_ref[...] * t + be_ref[...], 0.0)
    o_ref[...] = jnp.dot(h, w1n_ref[...], preferred_element_type=jnp.float32
                         ).astype(jnp.bfloat16)


def _mlp_block_tc(y, acc, em, b1, W2, b2, g, be, W1n):
    M, H = y.shape
    F = W1n.shape[1]
    grid = M // _ROWS
    row = lambda i: (i, 0)
    one = lambda i: (0, 0)
    return pl.pallas_call(
        _block_body,
        grid=(grid,),
        in_specs=[
            pl.BlockSpec((_ROWS, H), row),
            pl.BlockSpec((2, _ROWS, H), lambda i: (0, i, 0)),
            pl.BlockSpec((1, H), one),
            pl.BlockSpec((1, H), one),
            pl.BlockSpec((H, H), one),
            pl.BlockSpec((1, H), one),
            pl.BlockSpec((1, H), one),
            pl.BlockSpec((1, H), one),
            pl.BlockSpec((H, F), one),
        ],
        out_specs=pl.BlockSpec((_ROWS, F), row),
        out_shape=jax.ShapeDtypeStruct((M, F), jnp.bfloat16),
    )(y, acc, em, b1, W2, b2, g, be, W1n)


def _final_body(n_real, y_ref, a_ref, em_ref, b1_ref, w2_ref,
                b2_ref, g_ref, be_ref, o_ref, acc_ref):
    i = pl.program_id(0)
    agg = a_ref[0].astype(jnp.float32) + a_ref[1].astype(jnp.float32)
    z = em_ref[...] * y_ref[...].astype(jnp.float32) + agg + b1_ref[...]
    z = jnp.maximum(z, 0.0)
    rowid = lax.broadcasted_iota(jnp.int32, z.shape, 0) + i * _ROWS
    z = jnp.where(rowid < n_real, z, 0.0)
    part = jnp.sum(z, axis=0, keepdims=True)

    @pl.when(i == 0)
    def _():
        acc_ref[...] = jnp.zeros_like(acc_ref)

    acc_ref[...] += part

    @pl.when(i == pl.num_programs(0) - 1)
    def _():
        s = acc_ref[...] * (1.0 / n_real)
        t = jnp.dot(s, w2_ref[...], preferred_element_type=jnp.float32)
        o_ref[...] = g_ref[...] * (t + b2_ref[...]) + be_ref[...]


def _final_tc(y, acc, em, b1, W2, b2, g, be, n_real):
    M, F = y.shape
    grid = M // _ROWS
    row = lambda i: (i, 0)
    one = lambda i: (0, 0)
    return pl.pallas_call(
        functools.partial(_final_body, n_real),
        grid=(grid,),
        in_specs=[
            pl.BlockSpec((_ROWS, F), row),
            pl.BlockSpec((2, _ROWS, F), lambda i: (0, i, 0)),
            pl.BlockSpec((1, F), one),
            pl.BlockSpec((1, F), one),
            pl.BlockSpec((F, F), one),
            pl.BlockSpec((1, F), one),
            pl.BlockSpec((1, F), one),
            pl.BlockSpec((1, F), one),
        ],
        out_specs=pl.BlockSpec((1, F), one),
        out_shape=jax.ShapeDtypeStruct((1, F), jnp.float32),
        scratch_shapes=[pltpu.VMEM((1, F), jnp.float32)],
    )(y, acc, em, b1, W2, b2, g, be)


def _round_up(v, m):
    return (v + m - 1) // m * m


def kernel(x, edge_index, W1_0, b1_0, W2_0, b2_0, eps0, gamma0, beta0,
           W1_1, b1_1, W2_1, b2_1, eps1, gamma1, beta1):
    N, _ = x.shape
    H = W1_0.shape[1]
    OUT = W1_1.shape[1]
    E = edge_index.shape[1]
    NW = _NC * _NS

    del NW
    n_pad = _round_up(N, _ROWS)
    assert E % _BATCH == 0, "edge count must divide the 128-edge batch"
    eidx = edge_index.reshape(2, E // _BATCH, _BATCH)

    bn_scale = 1.0 / jnp.sqrt(1.0 + _BN_EPS)
    em0 = (1.0 + eps0) * jnp.ones((1, H), jnp.float32)
    em1 = (1.0 + eps1) * jnp.ones((1, OUT), jnp.float32)
    g0 = (gamma0 * bn_scale).reshape(1, H)
    g1 = (gamma1 * bn_scale).reshape(1, OUT)

    y0 = _linear_tc(x, W1_0, n_pad)
    acc0 = _segment_sum_sc(y0, eidx, n_pad, H)
    y1 = _mlp_block_tc(y0, acc0, em0, b1_0.reshape(1, H),
                       W2_0, b2_0.reshape(1, H), g0, beta0.reshape(1, H),
                       W1_1)
    acc1 = _segment_sum_sc(y1, eidx, n_pad, OUT)
    return _final_tc(y1, acc1, em1, b1_1.reshape(1, OUT),
                     W2_1, b2_1.reshape(1, OUT), g1, beta1.reshape(1, OUT),
                     N)


# speedup vs baseline: 3.1214x; 1.1202x over previous
import functools

import jax
import jax.numpy as jnp
from jax import lax
from jax.experimental import pallas as pl
from jax.experimental.pallas import tpu as pltpu
from jax.experimental.pallas import tpu_sc as plsc

_BN_EPS = 1e-5
_NC = 2
_NS = 16
_BATCH = 128
_ROWS = 2000
_ZR = 25


def _mm_body(x_ref, w_ref, o_ref):
    o_ref[...] = jnp.dot(x_ref[...], w_ref[...],
                         preferred_element_type=jnp.float32
                         ).astype(jnp.bfloat16)


def _linear_tc(x, W, m_out):
    _, K = x.shape
    F = W.shape[1]
    grid = m_out // _ROWS
    return pl.pallas_call(
        _mm_body,
        grid=(grid,),
        in_specs=[
            pl.BlockSpec((_ROWS, K), lambda i: (i, 0)),
            pl.BlockSpec((K, F), lambda i: (0, 0)),
        ],
        out_specs=pl.BlockSpec((_ROWS, F), lambda i: (i, 0)),
        out_shape=jax.ShapeDtypeStruct((m_out, F), jnp.bfloat16),
    )(x, W)


def _segment_sum_sc(y, eidx, n_pad, feat):
    rows_per_tile = n_pad // _NS
    nbat = eidx.shape[1]
    nw = _NC * _NS
    q, rem = divmod(nbat, nw)

    def body(y_hbm, eidx_hbm, out_hbm, src_v, dst_v, rows0_v,
             rows1_v, zbuf_v, y_sh, acc_sh, sem0, sem1, semz, semy):
        c = lax.axis_index("c")
        s = lax.axis_index("s")
        wid = s * _NC + c
        for r in range(_ZR):
            for qq in range(feat // 32):
                zbuf_v[r, pl.ds(qq * 32, 32)] = jnp.zeros((32,), jnp.bfloat16)
        base = s * rows_per_tile
        nz = rows_per_tile // _ZR
        for i in range(nz):
            pltpu.async_copy(zbuf_v, acc_sh.at[pl.ds(base + i * _ZR, _ZR)],
                             semz)
        pltpu.async_copy(y_hbm.at[pl.ds(base, rows_per_tile)],
                         y_sh.at[pl.ds(base, rows_per_tile)], semy)
        extra = (wid < rem).astype(jnp.int32)
        nb_w = q + extra
        base_b = wid * q + jnp.minimum(wid, rem)
        pltpu.async_copy(eidx_hbm.at[0, pl.ds(base_b, q)],
                         src_v.at[pl.ds(0, q)], sem0)
        pltpu.async_copy(eidx_hbm.at[1, pl.ds(base_b, q)],
                         dst_v.at[pl.ds(0, q)], sem1)

        @pl.when(extra == 1)
        def _():
            pltpu.async_copy(eidx_hbm.at[0, pl.ds(base_b + q, 1)],
                             src_v.at[pl.ds(q, 1)], sem0)
            pltpu.async_copy(eidx_hbm.at[1, pl.ds(base_b + q, 1)],
                             dst_v.at[pl.ds(q, 1)], sem1)

        for i in range(nz):
            pltpu.make_async_copy(
                zbuf_v, acc_sh.at[pl.ds(base + i * _ZR, _ZR)], semz).wait()
        pltpu.make_async_copy(
            y_hbm.at[pl.ds(base, rows_per_tile)],
            y_sh.at[pl.ds(base, rows_per_tile)], semy).wait()
        pltpu.make_async_copy(eidx_hbm.at[0, pl.ds(base_b, q)],
                              src_v.at[pl.ds(0, q)], sem0).wait()
        pltpu.make_async_copy(eidx_hbm.at[1, pl.ds(base_b, q)],
                              dst_v.at[pl.ds(0, q)], sem1).wait()

        @pl.when(extra == 1)
        def _():
            pltpu.make_async_copy(eidx_hbm.at[0, pl.ds(base_b + q, 1)],
                                  src_v.at[pl.ds(q, 1)], sem0).wait()
            pltpu.make_async_copy(eidx_hbm.at[1, pl.ds(base_b + q, 1)],
                                  dst_v.at[pl.ds(q, 1)], sem1).wait()

        plsc.subcore_barrier()

        def fire(j, buf, sem):
            pltpu.async_copy(y_sh.at[src_v.at[j]], buf, sem)

        def drain(j, buf, sem):
            pltpu.make_async_copy(y_sh.at[src_v.at[j]], buf, sem).wait()

        def scat(j, buf):
            pltpu.sync_copy(buf, acc_sh.at[dst_v.at[j]], add=True)

        def step(j, buf, sem, obuf, osem):
            @pl.when(j + 1 < nb_w)
            def _():
                fire(j + 1, obuf, osem)

            drain(j, buf, sem)
            scat(j, buf)

        fire(0, rows0_v, sem0)

        def eloop2(p, carry):
            j = 2 * p
            step(j, rows0_v, sem0, rows1_v, sem1)
            step(j + 1, rows1_v, sem1, rows0_v, sem0)
            return carry

        lax.fori_loop(0, nb_w // 2, eloop2, 0)

        @pl.when(nb_w % 2 == 1)
        def _():
            step(nb_w - 1, rows0_v, sem0, rows1_v, sem1)

        plsc.subcore_barrier()
        pltpu.sync_copy(acc_sh.at[pl.ds(base, rows_per_tile)],
                        out_hbm.at[c, pl.ds(base, rows_per_tile)])

    k = pl.kernel(
        body,
        out_type=jax.ShapeDtypeStruct((_NC, n_pad, feat), jnp.bfloat16),
        mesh=plsc.VectorSubcoreMesh(core_axis_name="c", subcore_axis_name="s"),
        scratch_types=[
            pltpu.VMEM((q + 1, _BATCH), jnp.int32),
            pltpu.VMEM((q + 1, _BATCH), jnp.int32),
            pltpu.VMEM((_BATCH, feat), jnp.bfloat16),
            pltpu.VMEM((_BATCH, feat), jnp.bfloat16),
            pltpu.VMEM((_ZR, feat), jnp.bfloat16),
            pltpu.VMEM_SHARED((n_pad, feat), jnp.bfloat16),
            pltpu.VMEM_SHARED((n_pad, feat), jnp.bfloat16),
            pltpu.SemaphoreType.DMA,
            pltpu.SemaphoreType.DMA,
            pltpu.SemaphoreType.DMA,
            pltpu.SemaphoreType.DMA,
        ],
        compiler_params=pltpu.CompilerParams(use_tc_tiling_on_sc=False),
    )
    return k(y, eidx)


def _block_body(y_ref, a_ref, em_ref, b1_ref, w2_ref, b2_ref,
                g_ref, be_ref, w1n_ref, o_ref):
    agg = a_ref[0].astype(jnp.float32) + a_ref[1].astype(jnp.float32)
    z = em_ref[...] * y_ref[...].astype(jnp.float32) + agg + b1_ref[...]
    z = jnp.maximum(z, 0.0)
    t = jnp.dot(z, w2_ref[...], preferred_element_type=jnp.float32) + b2_ref[...]
    h = jnp.maximum(g_ref[...] * t + be_ref[...], 0.0)
    o_ref[...] = jnp.dot(h, w1n_ref[...], preferred_element_type=jnp.float32
                         ).astype(jnp.bfloat16)


def _mlp_block_tc(y, acc, em, b1, W2, b2, g, be, W1n):
    M, H = y.shape
    F = W1n.shape[1]
    grid = M // _ROWS
    row = lambda i: (i, 0)
    one = lambda i: (0, 0)
    return pl.pallas_call(
        _block_body,
        grid=(grid,),
        in_specs=[
            pl.BlockSpec((_ROWS, H), row),
            pl.BlockSpec((2, _ROWS, H), lambda i: (0, i, 0)),
            pl.BlockSpec((1, H), one),
            pl.BlockSpec((1, H), one),
            pl.BlockSpec((H, H), one),
            pl.BlockSpec((1, H), one),
            pl.BlockSpec((1, H), one),
            pl.BlockSpec((1, H), one),
            pl.BlockSpec((H, F), one),
        ],
        out_specs=pl.BlockSpec((_ROWS, F), row),
        out_shape=jax.ShapeDtypeStruct((M, F), jnp.bfloat16),
    )(y, acc, em, b1, W2, b2, g, be, W1n)


def _final_body(n_real, y_ref, a_ref, em_ref, b1_ref, w2_ref,
                b2_ref, g_ref, be_ref, o_ref, acc_ref):
    i = pl.program_id(0)
    agg = a_ref[0].astype(jnp.float32) + a_ref[1].astype(jnp.float32)
    z = em_ref[...] * y_ref[...].astype(jnp.float32) + agg + b1_ref[...]
    z = jnp.maximum(z, 0.0)
    rowid = lax.broadcasted_iota(jnp.int32, z.shape, 0) + i * _ROWS
    z = jnp.where(rowid < n_real, z, 0.0)
    part = jnp.sum(z, axis=0, keepdims=True)

    @pl.when(i == 0)
    def _():
        acc_ref[...] = jnp.zeros_like(acc_ref)

    acc_ref[...] += part

    @pl.when(i == pl.num_programs(0) - 1)
    def _():
        s = acc_ref[...] * (1.0 / n_real)
        t = jnp.dot(s, w2_ref[...], preferred_element_type=jnp.float32)
        o_ref[...] = g_ref[...] * (t + b2_ref[...]) + be_ref[...]


def _final_tc(y, acc, em, b1, W2, b2, g, be, n_real):
    M, F = y.shape
    grid = M // _ROWS
    row = lambda i: (i, 0)
    one = lambda i: (0, 0)
    return pl.pallas_call(
        functools.partial(_final_body, n_real),
        grid=(grid,),
        in_specs=[
            pl.BlockSpec((_ROWS, F), row),
            pl.BlockSpec((2, _ROWS, F), lambda i: (0, i, 0)),
            pl.BlockSpec((1, F), one),
            pl.BlockSpec((1, F), one),
            pl.BlockSpec((F, F), one),
            pl.BlockSpec((1, F), one),
            pl.BlockSpec((1, F), one),
            pl.BlockSpec((1, F), one),
        ],
        out_specs=pl.BlockSpec((1, F), one),
        out_shape=jax.ShapeDtypeStruct((1, F), jnp.float32),
        scratch_shapes=[pltpu.VMEM((1, F), jnp.float32)],
    )(y, acc, em, b1, W2, b2, g, be)


def _round_up(v, m):
    return (v + m - 1) // m * m


def kernel(x, edge_index, W1_0, b1_0, W2_0, b2_0, eps0, gamma0, beta0,
           W1_1, b1_1, W2_1, b2_1, eps1, gamma1, beta1):
    N, _ = x.shape
    H = W1_0.shape[1]
    OUT = W1_1.shape[1]
    E = edge_index.shape[1]
    NW = _NC * _NS

    del NW
    assert N % _ROWS == 0 and (N // _NS) % _ZR == 0
    n_pad = N
    assert E % _BATCH == 0, "edge count must divide the 128-edge batch"
    eidx = edge_index.reshape(2, E // _BATCH, _BATCH)

    bn_scale = 1.0 / jnp.sqrt(1.0 + _BN_EPS)
    em0 = (1.0 + eps0) * jnp.ones((1, H), jnp.float32)
    em1 = (1.0 + eps1) * jnp.ones((1, OUT), jnp.float32)
    g0 = (gamma0 * bn_scale).reshape(1, H)
    g1 = (gamma1 * bn_scale).reshape(1, OUT)

    y0 = _linear_tc(x, W1_0, n_pad)
    acc0 = _segment_sum_sc(y0, eidx, n_pad, H)
    y1 = _mlp_block_tc(y0, acc0, em0, b1_0.reshape(1, H),
                       W2_0, b2_0.reshape(1, H), g0, beta0.reshape(1, H),
                       W1_1)
    acc1 = _segment_sum_sc(y1, eidx, n_pad, OUT)
    return _final_tc(y1, acc1, em1, b1_1.reshape(1, OUT),
                     W2_1, b2_1.reshape(1, OUT), g1, beta1.reshape(1, OUT),
                     N)


# speedup vs baseline: 3.1997x; 1.0251x over previous
import functools

import jax
import jax.numpy as jnp
from jax import lax
from jax.experimental import pallas as pl
from jax.experimental.pallas import tpu as pltpu
from jax.experimental.pallas import tpu_sc as plsc

_BN_EPS = 1e-5
_NC = 2
_NS = 16
_BATCH = 128
_BROWS = 1000
_ZR = 25


def _mm_packed_body(xt_ref, xb_ref, w_ref, o_ref):
    f = w_ref.shape[1]
    o_ref[:, :f] = jnp.dot(xt_ref[...], w_ref[...],
                           preferred_element_type=jnp.float32
                           ).astype(jnp.bfloat16)
    o_ref[:, f:] = jnp.dot(xb_ref[...], w_ref[...],
                           preferred_element_type=jnp.float32
                           ).astype(jnp.bfloat16)


def _linear_packed_tc(x, W):
    M, K = x.shape
    F = W.shape[1]
    half = M // 2
    grid = half // _BROWS
    return pl.pallas_call(
        _mm_packed_body,
        grid=(grid,),
        in_specs=[
            pl.BlockSpec((_BROWS, K), lambda i: (i, 0)),
            pl.BlockSpec((_BROWS, K), lambda i, g=grid: (i + g, 0)),
            pl.BlockSpec((K, F), lambda i: (0, 0)),
        ],
        out_specs=pl.BlockSpec((_BROWS, 2 * F), lambda i: (i, 0)),
        out_shape=jax.ShapeDtypeStruct((half, 2 * F), jnp.bfloat16),
    )(x, x, W)


def _segment_sum_sc(y, eidx, n_pad, feat):
    rows_per_tile = n_pad // _NS
    nbat = eidx.shape[1]
    nw = _NC * _NS
    q, rem = divmod(nbat, nw)

    def body(y_hbm, eidx_hbm, out_hbm, src_v, dst_v, rows0_v,
             rows1_v, zbuf_v, y_sh, acc_sh, sem0, sem1, semz, semy):
        c = lax.axis_index("c")
        s = lax.axis_index("s")
        wid = s * _NC + c
        for r in range(_ZR):
            for qq in range(feat // 32):
                zbuf_v[r, pl.ds(qq * 32, 32)] = jnp.zeros((32,), jnp.bfloat16)
        base = s * rows_per_tile
        nz = rows_per_tile // _ZR
        for i in range(nz):
            pltpu.async_copy(zbuf_v, acc_sh.at[pl.ds(base + i * _ZR, _ZR)],
                             semz)
        pltpu.async_copy(y_hbm.at[pl.ds(base, rows_per_tile)],
                         y_sh.at[pl.ds(base, rows_per_tile)], semy)
        extra = (wid < rem).astype(jnp.int32)
        nb_w = q + extra
        base_b = wid * q + jnp.minimum(wid, rem)
        pltpu.async_copy(eidx_hbm.at[0, pl.ds(base_b, q)],
                         src_v.at[pl.ds(0, q)], sem0)
        pltpu.async_copy(eidx_hbm.at[1, pl.ds(base_b, q)],
                         dst_v.at[pl.ds(0, q)], sem1)

        @pl.when(extra == 1)
        def _():
            pltpu.async_copy(eidx_hbm.at[0, pl.ds(base_b + q, 1)],
                             src_v.at[pl.ds(q, 1)], sem0)
            pltpu.async_copy(eidx_hbm.at[1, pl.ds(base_b + q, 1)],
                             dst_v.at[pl.ds(q, 1)], sem1)

        for i in range(nz):
            pltpu.make_async_copy(
                zbuf_v, acc_sh.at[pl.ds(base + i * _ZR, _ZR)], semz).wait()
        pltpu.make_async_copy(
            y_hbm.at[pl.ds(base, rows_per_tile)],
            y_sh.at[pl.ds(base, rows_per_tile)], semy).wait()
        pltpu.make_async_copy(eidx_hbm.at[0, pl.ds(base_b, q)],
                              src_v.at[pl.ds(0, q)], sem0).wait()
        pltpu.make_async_copy(eidx_hbm.at[1, pl.ds(base_b, q)],
                              dst_v.at[pl.ds(0, q)], sem1).wait()

        @pl.when(extra == 1)
        def _():
            pltpu.make_async_copy(eidx_hbm.at[0, pl.ds(base_b + q, 1)],
                                  src_v.at[pl.ds(q, 1)], sem0).wait()
            pltpu.make_async_copy(eidx_hbm.at[1, pl.ds(base_b + q, 1)],
                                  dst_v.at[pl.ds(q, 1)], sem1).wait()

        plsc.subcore_barrier()

        def fire(j, buf, sem):
            pltpu.async_copy(y_sh.at[src_v.at[j]], buf, sem)

        def drain(j, buf, sem):
            pltpu.make_async_copy(y_sh.at[src_v.at[j]], buf, sem).wait()

        def scat(j, buf):
            pltpu.sync_copy(buf, acc_sh.at[dst_v.at[j]], add=True)

        def step(j, buf, sem, obuf, osem):
            @pl.when(j + 1 < nb_w)
            def _():
                fire(j + 1, obuf, osem)

            drain(j, buf, sem)
            scat(j, buf)

        fire(0, rows0_v, sem0)

        def eloop2(p, carry):
            j = 2 * p
            step(j, rows0_v, sem0, rows1_v, sem1)
            step(j + 1, rows1_v, sem1, rows0_v, sem0)
            return carry

        lax.fori_loop(0, nb_w // 2, eloop2, 0)

        @pl.when(nb_w % 2 == 1)
        def _():
            step(nb_w - 1, rows0_v, sem0, rows1_v, sem1)

        plsc.subcore_barrier()
        pltpu.sync_copy(acc_sh.at[pl.ds(base, rows_per_tile)],
                        out_hbm.at[c, pl.ds(base, rows_per_tile)])

    k = pl.kernel(
        body,
        out_type=jax.ShapeDtypeStruct((_NC, n_pad, feat), jnp.bfloat16),
        mesh=plsc.VectorSubcoreMesh(core_axis_name="c", subcore_axis_name="s"),
        scratch_types=[
            pltpu.VMEM((q + 1, _BATCH), jnp.int32),
            pltpu.VMEM((q + 1, _BATCH), jnp.int32),
            pltpu.VMEM((_BATCH, feat), jnp.bfloat16),
            pltpu.VMEM((_BATCH, feat), jnp.bfloat16),
            pltpu.VMEM((_ZR, feat), jnp.bfloat16),
            pltpu.VMEM_SHARED((n_pad, feat), jnp.bfloat16),
            pltpu.VMEM_SHARED((n_pad, feat), jnp.bfloat16),
            pltpu.SemaphoreType.DMA,
            pltpu.SemaphoreType.DMA,
            pltpu.SemaphoreType.DMA,
            pltpu.SemaphoreType.DMA,
        ],
        compiler_params=pltpu.CompilerParams(use_tc_tiling_on_sc=False),
    )
    return k(y, eidx)


def _block_packed_body(y_ref, a_ref, em_ref, b1_ref, w2_ref, b2_ref,
                       g_ref, be_ref, m1_ref, m2_ref, o_ref):
    agg = a_ref[0].astype(jnp.float32) + a_ref[1].astype(jnp.float32)
    z = em_ref[...] * y_ref[...].astype(jnp.float32) + agg + b1_ref[...]
    z = jnp.maximum(z, 0.0)
    t = jnp.dot(z, w2_ref[...], preferred_element_type=jnp.float32) + b2_ref[...]
    h = jnp.maximum(g_ref[...] * t + be_ref[...], 0.0)
    half = h.shape[0] // 2
    o_ref[...] = (jnp.dot(h[:half], m1_ref[...],
                          preferred_element_type=jnp.float32)
                  + jnp.dot(h[half:], m2_ref[...],
                            preferred_element_type=jnp.float32)
                  ).astype(jnp.bfloat16)


def _mlp_packed_tc(y, acc, em, b1, W2bd, b2, g, be, M1, M2):
    R, P = y.shape
    whole3 = lambda i: (0, 0, 0)
    one = lambda i: (0, 0)
    return pl.pallas_call(
        _block_packed_body,
        grid=(1,),
        in_specs=[
            pl.BlockSpec((R, P), one),
            pl.BlockSpec((2, R, P), whole3),
            pl.BlockSpec((1, P), one),
            pl.BlockSpec((1, P), one),
            pl.BlockSpec((P, P), one),
            pl.BlockSpec((1, P), one),
            pl.BlockSpec((1, P), one),
            pl.BlockSpec((1, P), one),
            pl.BlockSpec((P, P), one),
            pl.BlockSpec((P, P), one),
        ],
        out_specs=pl.BlockSpec((R // 2, P), one),
        out_shape=jax.ShapeDtypeStruct((R // 2, P), jnp.bfloat16),
    )(y, acc, em, b1, W2bd, b2, g, be, M1, M2)


def _final_packed_body(n_real, nf, y_ref, a_ref, em_ref, b1_ref,
                       w2_ref, b2_ref, g_ref, be_ref, o_ref):
    agg = a_ref[0].astype(jnp.float32) + a_ref[1].astype(jnp.float32)
    z = em_ref[...] * y_ref[...].astype(jnp.float32) + agg + b1_ref[...]
    z = jnp.maximum(z, 0.0)
    s = jnp.sum(z, axis=0, keepdims=True)
    s32 = (s[:, :nf] + s[:, nf:2 * nf] + s[:, 2 * nf:3 * nf]
           + s[:, 3 * nf:]) * (1.0 / n_real)
    t = jnp.dot(s32, w2_ref[...], preferred_element_type=jnp.float32)
    o_ref[...] = g_ref[...] * (t + b2_ref[...]) + be_ref[...]


def _final_packed_tc(y, acc, em, b1, W2, b2, g, be, n_real):
    R, P = y.shape
    nf = P // 4
    whole3 = lambda i: (0, 0, 0)
    one = lambda i: (0, 0)
    return pl.pallas_call(
        functools.partial(_final_packed_body, n_real, nf),
        grid=(1,),
        in_specs=[
            pl.BlockSpec((R, P), one),
            pl.BlockSpec((2, R, P), whole3),
            pl.BlockSpec((1, P), one),
            pl.BlockSpec((1, P), one),
            pl.BlockSpec((nf, nf), one),
            pl.BlockSpec((1, nf), one),
            pl.BlockSpec((1, nf), one),
            pl.BlockSpec((1, nf), one),
        ],
        out_specs=pl.BlockSpec((1, nf), one),
        out_shape=jax.ShapeDtypeStruct((1, nf), jnp.float32),
    )(y, acc, em, b1, W2, b2, g, be)


def _round_up(v, m):
    return (v + m - 1) // m * m


def kernel(x, edge_index, W1_0, b1_0, W2_0, b2_0, eps0, gamma0, beta0,
           W1_1, b1_1, W2_1, b2_1, eps1, gamma1, beta1):
    N, _ = x.shape
    H = W1_0.shape[1]
    OUT = W1_1.shape[1]
    E = edge_index.shape[1]

    assert N % 4 == 0 and (N // _NS) % _ZR == 0 and (N // 2) % _BROWS == 0
    assert E % _BATCH == 0, "edge count must divide the 128-edge batch"
    h2 = N // 2
    q4 = N // 4

    e = edge_index
    p0 = jnp.where(e < h2, 2 * e, 2 * (e - h2) + 1)
    p1 = 4 * (e % q4) + e // q4
    eidx0 = p0.reshape(2, E // _BATCH, _BATCH)
    eidx1 = p1.reshape(2, E // _BATCH, _BATCH)

    bn_scale = 1.0 / jnp.sqrt(1.0 + _BN_EPS)
    em0 = (1.0 + eps0) * jnp.ones((1, 2 * H), jnp.float32)
    em1 = (1.0 + eps1) * jnp.ones((1, 4 * OUT), jnp.float32)
    b1_0p = jnp.tile(b1_0, 2).reshape(1, 2 * H)
    b2_0p = jnp.tile(b2_0, 2).reshape(1, 2 * H)
    g0p = jnp.tile(gamma0 * bn_scale, 2).reshape(1, 2 * H)
    be0p = jnp.tile(beta0, 2).reshape(1, 2 * H)
    b1_1p = jnp.tile(b1_1, 4).reshape(1, 4 * OUT)
    g1 = (gamma1 * bn_scale).reshape(1, OUT)
    zH = jnp.zeros((H, H), jnp.float32)
    W2bd = jnp.block([[W2_0, zH], [zH, W2_0]])
    zHO = jnp.zeros((H, OUT), jnp.float32)
    M1 = jnp.block([[W1_1, zHO, zHO, zHO], [zHO, zHO, W1_1, zHO]])
    M2 = jnp.block([[zHO, W1_1, zHO, zHO], [zHO, zHO, zHO, W1_1]])

    y0p = _linear_packed_tc(x, W1_0)
    acc0 = _segment_sum_sc(jnp.reshape(y0p, (N, H)), eidx0, N, H)
    acc0p = jnp.reshape(acc0, (2, h2, 2 * H))
    y1p = _mlp_packed_tc(y0p, acc0p, em0, b1_0p, W2bd, b2_0p, g0p, be0p,
                         M1, M2)
    acc1 = _segment_sum_sc(jnp.reshape(y1p, (N, OUT)), eidx1, N, OUT)
    acc1p = jnp.reshape(acc1, (2, q4, 4 * OUT))
    return _final_packed_tc(y1p, acc1p, em1, b1_1p, W2_1,
                            b2_1.reshape(1, OUT), g1, beta1.reshape(1, OUT),
                            N)
